# baseline (speedup 1.0000x reference)
import functools

import jax
import jax.numpy as jnp
from jax import lax
from jax.experimental import pallas as pl
from jax.experimental.pallas import tpu as pltpu
from jax.experimental.pallas import tpu_sc as plsc

VOCAB = 100000
EMBED_DIM = 64
BATCH = 4096
HIST = 50

NUM_CORES = 2
NUM_SUBCORES = 16
NUM_WORKERS = NUM_CORES * NUM_SUBCORES
COLS_PER_WORKER = EMBED_DIM // NUM_WORKERS

LANES = 16
UNROLL = 4
INNER = BATCH // (LANES * UNROLL)


_mesh = plsc.VectorSubcoreMesh(core_axis_name="c", subcore_axis_name="s")


@functools.partial(
    pl.kernel,
    out_type=jax.ShapeDtypeStruct((HIST, EMBED_DIM, BATCH), jnp.float32),
    mesh=_mesh,
    scratch_types=(
        [pltpu.VMEM((VOCAB,), jnp.float32)]
        + [pltpu.VMEM((BATCH,), jnp.int32) for _ in range(2)]
        + [pltpu.VMEM((BATCH,), jnp.float32) for _ in range(2)]
        + [pltpu.SemaphoreType.DMA for _ in range(4)]
    ),
    compiler_params=pltpu.CompilerParams(
        use_tc_tiling_on_sc=False, needs_layout_passes=False),
)
def _lookup_kernel(ids_hbm, table_hbm, out_hbm,
                   row_v, ids0, ids1, outv0, outv1,
                   si0, si1, so0, so1):
    wid = lax.axis_index("s") * NUM_CORES + lax.axis_index("c")

    ids_v = (ids0, ids1)
    out_v = (outv0, outv1)
    si = (si0, si1)
    so = (so0, so1)

    def gather_batch(idsb, outb):
        def body(i, carry):
            for u in range(UNROLL):
                o = (i * UNROLL + u) * LANES
                idx = idsb[pl.ds(o, LANES)]
                outb[pl.ds(o, LANES)] = plsc.load_gather(row_v, [idx])
            return carry
        lax.fori_loop(0, INNER, body, 0)

    for k in range(COLS_PER_WORKER):
        col = wid * COLS_PER_WORKER + k
        pltpu.sync_copy(table_hbm.at[col], row_v)
        ids_d, out_d = {}, {}
        ids_d[0] = pltpu.async_copy(ids_hbm.at[0], ids_v[0], si[0])
        for h in range(HIST):
            b = h % 2
            ids_d[h].wait()
            if h + 1 < HIST:
                ids_d[h + 1] = pltpu.async_copy(
                    ids_hbm.at[h + 1], ids_v[1 - b], si[1 - b])
            if h - 2 >= 0:
                out_d[h - 2].wait()
            gather_batch(ids_v[b], out_v[b])
            out_d[h] = pltpu.async_copy(out_v[b], out_hbm.at[h, col], so[b])
        out_d[HIST - 2].wait()
        out_d[HIST - 1].wait()


def kernel(start_ids, table):
    ids_t = start_ids.T.astype(jnp.int32)
    table_t = table.T
    out_t = _lookup_kernel(ids_t, table_t)
    return jnp.transpose(out_t, (2, 0, 1))


# speedup vs baseline: 1.0131x; 1.0131x over previous
import functools

import jax
import jax.numpy as jnp
from jax import lax
from jax.experimental import pallas as pl
from jax.experimental.pallas import tpu as pltpu
from jax.experimental.pallas import tpu_sc as plsc

VOCAB = 100000
EMBED_DIM = 64
BATCH = 4096
HIST = 50

NUM_CORES = 2
NUM_SUBCORES = 16
NUM_WORKERS = NUM_CORES * NUM_SUBCORES

TOTAL = BATCH * HIST
PER_WORKER = TOTAL // NUM_WORKERS
CHUNK = 800
NUM_CHUNKS = PER_WORKER // CHUNK
NBUF = 2


_mesh = plsc.VectorSubcoreMesh(core_axis_name="c", subcore_axis_name="s")


@functools.partial(
    pl.kernel,
    out_type=jax.ShapeDtypeStruct((TOTAL, EMBED_DIM), jnp.float32),
    mesh=_mesh,
    scratch_types=(
        [pltpu.VMEM((CHUNK,), jnp.int32) for _ in range(NBUF)]
        + [pltpu.VMEM((CHUNK, EMBED_DIM), jnp.float32) for _ in range(NBUF)]
        + [pltpu.SemaphoreType.DMA for _ in range(3 * NBUF)]
    ),
    compiler_params=pltpu.CompilerParams(
        use_tc_tiling_on_sc=False, needs_layout_passes=False),
)
def _gather_kernel(ids_hbm, table_hbm, out_hbm,
                   idx0, idx1, rows0, rows1,
                   si0, si1, sg0, sg1, ss0, ss1):
    wid = lax.axis_index("s") * NUM_CORES + lax.axis_index("c")
    base = wid * PER_WORKER

    idx = (idx0, idx1)
    rows = (rows0, rows1)
    si = (si0, si1)
    sg = (sg0, sg1)
    ss = (ss0, ss1)

    def off(c):
        return base + c * CHUNK

    idx_d, g_d, s_d = {}, {}, {}
    for c in range(min(NBUF, NUM_CHUNKS)):
        b = c % NBUF
        idx_d[c] = pltpu.async_copy(
            ids_hbm.at[pl.ds(off(c), CHUNK)], idx[b], si[b])
    idx_d[0].wait()
    g_d[0] = pltpu.async_copy(table_hbm.at[idx[0]], rows[0], sg[0])

    for c in range(NUM_CHUNKS):
        b = c % NBUF
        b2 = (c + 1) % NBUF
        if c + 1 < NUM_CHUNKS:
            idx_d[c + 1].wait()
            if c - 1 >= 0:
                s_d[c - 1].wait()
            g_d[c + 1] = pltpu.async_copy(
                table_hbm.at[idx[b2]], rows[b2], sg[b2])
        g_d[c].wait()
        s_d[c] = pltpu.async_copy(
            rows[b], out_hbm.at[pl.ds(off(c), CHUNK)], ss[b])
        if c + 2 < NUM_CHUNKS:
            idx_d[c + 2] = pltpu.async_copy(
                ids_hbm.at[pl.ds(off(c + 2), CHUNK)], idx[b], si[b])

    if NUM_CHUNKS >= 2:
        s_d[NUM_CHUNKS - 2].wait()
    s_d[NUM_CHUNKS - 1].wait()


def kernel(start_ids, table):
    ids = start_ids.reshape(-1).astype(jnp.int32)
    out = _gather_kernel(ids, table)
    return out.reshape(BATCH, HIST, EMBED_DIM)


# speedup vs baseline: 1.0707x; 1.0568x over previous
import functools

import jax
import jax.numpy as jnp
from jax import lax
from jax.experimental import pallas as pl
from jax.experimental.pallas import tpu as pltpu
from jax.experimental.pallas import tpu_sc as plsc

VOCAB = 100000
EMBED_DIM = 64
BATCH = 4096
HIST = 50

NUM_CORES = 2
NUM_SUBCORES = 16
NUM_WORKERS = NUM_CORES * NUM_SUBCORES
COLS_PER_WORKER = EMBED_DIM // NUM_WORKERS

LANES = 16
UNROLL = 8
INNER = BATCH // (LANES * UNROLL)


_mesh = plsc.VectorSubcoreMesh(core_axis_name="c", subcore_axis_name="s")


@functools.partial(
    pl.kernel,
    out_type=jax.ShapeDtypeStruct((HIST, EMBED_DIM, BATCH), jnp.float32),
    mesh=_mesh,
    scratch_types=(
        [pltpu.VMEM((VOCAB,), jnp.float32)]
        + [pltpu.VMEM((BATCH,), jnp.int32) for _ in range(2)]
        + [pltpu.VMEM((BATCH,), jnp.float32) for _ in range(2)]
        + [pltpu.SemaphoreType.DMA for _ in range(4)]
    ),
    compiler_params=pltpu.CompilerParams(
        use_tc_tiling_on_sc=False, needs_layout_passes=False),
)
def _lookup_kernel(ids_hbm, table_hbm, out_hbm,
                   row_v, ids0, ids1, outv0, outv1,
                   si0, si1, so0, so1):
    wid = lax.axis_index("s") * NUM_CORES + lax.axis_index("c")

    ids_v = (ids0, ids1)
    out_v = (outv0, outv1)
    si = (si0, si1)
    so = (so0, so1)

    def gather_batch(idsb, outb):
        def body(i, carry):
            base = i * UNROLL * LANES
            idxs = [idsb[pl.ds(base + u * LANES, LANES)]
                    for u in range(UNROLL)]
            vals = [plsc.load_gather(row_v, [idxs[u]])
                    for u in range(UNROLL)]
            for u in range(UNROLL):
                outb[pl.ds(base + u * LANES, LANES)] = vals[u]
            return carry
        lax.fori_loop(0, INNER, body, 0)

    for k in range(COLS_PER_WORKER):
        col = wid * COLS_PER_WORKER + k
        pltpu.sync_copy(table_hbm.at[col], row_v)
        ids_d, out_d = {}, {}
        ids_d[0] = pltpu.async_copy(ids_hbm.at[0], ids_v[0], si[0])
        for h in range(HIST):
            b = h % 2
            ids_d[h].wait()
            if h + 1 < HIST:
                ids_d[h + 1] = pltpu.async_copy(
                    ids_hbm.at[h + 1], ids_v[1 - b], si[1 - b])
            if h - 2 >= 0:
                out_d[h - 2].wait()
            gather_batch(ids_v[b], out_v[b])
            out_d[h] = pltpu.async_copy(out_v[b], out_hbm.at[h, col], so[b])
        out_d[HIST - 2].wait()
        out_d[HIST - 1].wait()


def kernel(start_ids, table):
    ids_t = start_ids.T.astype(jnp.int32)
    table_t = table.T
    out_t = _lookup_kernel(ids_t, table_t)
    return jnp.transpose(out_t, (2, 0, 1))
